# baseline (speedup 1.0000x reference)
import jax
import jax.numpy as jnp
from jax.experimental import pallas as pl
from jax.experimental.pallas import tpu as pltpu

D = 100000
B = 128
CW = 8192
NCH = (D + CW - 1) // CW
LAST = D - (NCH - 1) * CW
NBUF = 3
LA = NBUF - 1
_I32MAX = jnp.iinfo(jnp.int32).max


def _in_copies(u_hbm, e_hbm, ub, eb, su, se, j):
    slot = jax.lax.rem(j, NBUF)
    cu = pltpu.make_async_copy(
        u_hbm.at[:, pl.ds(j * CW, CW)], ub.at[slot], su.at[slot])
    ce = pltpu.make_async_copy(
        e_hbm.at[:, pl.ds(j * CW, CW)], eb.at[slot], se.at[slot])
    return cu, ce


def _in_copies_last(u_hbm, e_hbm, ubl, ebl, sul, sel):
    base = (NCH - 1) * CW
    cu = pltpu.make_async_copy(u_hbm.at[:, pl.ds(base, LAST)], ubl, sul)
    ce = pltpu.make_async_copy(e_hbm.at[:, pl.ds(base, LAST)], ebl, sel)
    return cu, ce


def _out_copy(out_hbm, ob, so, j):
    slot = jax.lax.rem(j, NBUF)
    return pltpu.make_async_copy(
        ob.at[slot], out_hbm.at[:, pl.ds(j * CW, CW)], so.at[slot])


def _out_copy_last(out_hbm, obl, sol):
    base = (NCH - 1) * CW
    return pltpu.make_async_copy(
        obl, out_hbm.at[:, pl.ds(base, LAST + 1)], sol)


def _body(pb_ref, m_ref, ls_ref, pfull_ref, u_hbm, e_hbm,
          out_hbm, y_ref,
          ub, eb, ob, ubl, ebl, obl,
          su, se, so, sul, sel, sol,
          bk, bi, bp):
    i = pl.program_id(0)
    slot = jax.lax.rem(i, NBUF)

    @pl.when(i == 0)
    def _prologue():
        bk[...] = jnp.full((B, 1), jnp.inf, jnp.float32)
        bi[...] = jnp.zeros((B, 1), jnp.int32)
        bp[...] = jnp.zeros((B, 1), jnp.float32)
        for j in range(LA):
            cu, ce = _in_copies(u_hbm, e_hbm, ub, eb, su, se, j)
            cu.start()
            ce.start()

    j = i + LA

    @pl.when(j < NCH - 1)
    def _start_full():
        cu, ce = _in_copies(u_hbm, e_hbm, ub, eb, su, se, j)
        cu.start()
        ce.start()

    @pl.when(j == NCH - 1)
    def _start_last():
        cu, ce = _in_copies_last(u_hbm, e_hbm, ubl, ebl, sul, sel)
        cu.start()
        ce.start()

    @pl.when(jnp.logical_and(i >= NBUF, i < NCH - 1))
    def _drain_out():
        _out_copy(out_hbm, ob, so, i - NBUF).wait()

    def update(key, pb_c, lane):
        local_min = jnp.min(key, axis=1, keepdims=True)
        w = key == local_min
        local_arg = jnp.min(jnp.where(w, lane, _I32MAX),
                            axis=1, keepdims=True)
        local_prob = jnp.max(jnp.where(w, pb_c, -jnp.inf),
                             axis=1, keepdims=True)
        upd = local_min < bk[...]
        bk[...] = jnp.where(upd, local_min, bk[...])
        bi[...] = jnp.where(upd, i * CW + local_arg, bi[...])
        bp[...] = jnp.where(upd, local_prob, bp[...])

    @pl.when(i < NCH - 1)
    def _compute_full():
        cu, ce = _in_copies(u_hbm, e_hbm, ub, eb, su, se, i)
        cu.wait()
        ce.wait()
        pb = pb_ref[...]
        lane = jax.lax.broadcasted_iota(jnp.int32, (1, CW), 1)
        e = -jnp.log(ub[slot])
        key = e * jnp.exp(-pb)
        ob[slot] = m_ref[...] + jnp.exp(ls_ref[...]) * eb[slot]
        update(key, pb, lane)
        _out_copy(out_hbm, ob, so, i).start()

    @pl.when(i == NCH - 1)
    def _compute_last():
        cu, ce = _in_copies_last(u_hbm, e_hbm, ubl, ebl, sul, sel)
        cu.wait()
        ce.wait()
        pb = pb_ref[...][:, :LAST]
        lane = jax.lax.broadcasted_iota(jnp.int32, (1, LAST), 1)
        e = -jnp.log(ubl[...])
        key = e * jnp.exp(-pb)
        obl[:, :LAST] = (m_ref[...][:, :LAST]
                         + jnp.exp(ls_ref[...][:, :LAST]) * ebl[...])
        update(key, pb, lane)

        pf = pfull_ref[...]
        mx = jnp.max(pf, keepdims=True).reshape(1, 1)
        s = jnp.sum(jnp.exp(pf - mx), keepdims=True).reshape(1, 1)
        lse = mx + jnp.log(s)
        y_ref[...] = bi[...]
        obl[:, LAST:LAST + 1] = bp[...] - lse
        _out_copy_last(out_hbm, obl, sol).start()

        for k in range(NCH - 1 - NBUF, NCH - 1):
            _out_copy(out_hbm, ob, so, k).wait()
        _out_copy_last(out_hbm, obl, sol).wait()


@jax.jit
def kernel(prob, m_z, log_s_z, u, eps):
    prob2 = prob.reshape(1, D)
    m2 = m_z.reshape(1, D)
    ls2 = log_s_z.reshape(1, D)

    row_spec = pl.BlockSpec((1, CW), lambda i: (0, i))
    full_spec = pl.BlockSpec((1, D), lambda i: (0, 0))
    any_spec = pl.BlockSpec(memory_space=pltpu.MemorySpace.HBM)

    out, y2 = pl.pallas_call(
        _body,
        grid=(NCH,),
        in_specs=[row_spec, row_spec, row_spec, full_spec,
                  any_spec, any_spec],
        out_specs=[
            any_spec,
            pl.BlockSpec((B, 1), lambda i: (0, 0)),
        ],
        out_shape=[
            jax.ShapeDtypeStruct((B, D + 1), jnp.float32),
            jax.ShapeDtypeStruct((B, 1), jnp.int32),
        ],
        scratch_shapes=[
            pltpu.VMEM((NBUF, B, CW), jnp.float32),
            pltpu.VMEM((NBUF, B, CW), jnp.float32),
            pltpu.VMEM((NBUF, B, CW), jnp.float32),
            pltpu.VMEM((B, LAST), jnp.float32),
            pltpu.VMEM((B, LAST), jnp.float32),
            pltpu.VMEM((B, LAST + 1), jnp.float32),
            pltpu.SemaphoreType.DMA((NBUF,)),
            pltpu.SemaphoreType.DMA((NBUF,)),
            pltpu.SemaphoreType.DMA((NBUF,)),
            pltpu.SemaphoreType.DMA,
            pltpu.SemaphoreType.DMA,
            pltpu.SemaphoreType.DMA,
            pltpu.VMEM((B, 1), jnp.float32),
            pltpu.VMEM((B, 1), jnp.int32),
            pltpu.VMEM((B, 1), jnp.float32),
        ],
    )(prob2, m2, ls2, prob2, u, eps)
    return (y2.reshape(B), out)
